# baseline (speedup 1.0000x reference)
import functools

import jax
import jax.numpy as jnp
from jax import lax
from jax.experimental import pallas as pl
from jax.experimental.pallas import tpu as pltpu
from jax.experimental.pallas import tpu_sc as plsc

try:
    _info = plsc.get_sparse_core_info()
    NC, NS = _info.num_cores, _info.num_subcores
except Exception:
    NC, NS = 2, 16
NW = NC * NS

C = 128
BPT = 512
NBLK = BPT // C


@functools.cache
def _build(B, H, D):
    mesh = plsc.VectorSubcoreMesh(
        core_axis_name="c", subcore_axis_name="s", num_cores=NC, num_subcores=NS
    )

    @functools.partial(
        pl.kernel,
        mesh=mesh,
        out_type=jax.ShapeDtypeStruct((H, D, B), jnp.float32),
        scratch_types=[
            pltpu.VMEM((H, BPT), jnp.int32),
            pltpu.VMEM((C,), jnp.int32),
            pltpu.VMEM((C,), jnp.int32),
            pltpu.VMEM((C, 2 * D), jnp.float32),
            pltpu.VMEM((D, C), jnp.float32),
            pltpu.SemaphoreType.DMA,
        ],
        compiler_params=pltpu.CompilerParams(
            use_tc_tiling_on_sc=True, needs_layout_passes=False
        ),
    )
    def gk(idxT_hbm, tabP_hbm, out_hbm, idx_v, vp_v, hf_v, buf, bufT, sem):
        wid = lax.axis_index("s") * NC + lax.axis_index("c")
        bs = wid * BPT
        pltpu.sync_copy(idxT_hbm.at[:, pl.ds(bs, BPT)], idx_v)
        iota = lax.iota(jnp.int32, 16)

        def do_h(h, carry):
            for blk in range(NBLK):
                for g in range(C // 16):
                    seg = idx_v[h, pl.ds(blk * C + g * 16, 16)]
                    vp_v[pl.ds(g * 16, 16)] = seg >> 1
                    hf_v[pl.ds(g * 16, 16)] = (seg & 1) * D
                pltpu.async_copy(tabP_hbm.at[vp_v], buf, sem).wait()

                def do_d(d, c2):
                    for g in range(C // 16):
                        rows = iota + g * 16
                        cols = hf_v[pl.ds(g * 16, 16)] + d
                        vals = plsc.load_gather(buf, [rows, cols])
                        bufT[d, pl.ds(g * 16, 16)] = vals
                    return c2

                lax.fori_loop(0, D, do_d, 0, unroll=False)
                pltpu.sync_copy(bufT, out_hbm.at[h, :, pl.ds(bs + blk * C, C)])
            return carry

        lax.fori_loop(0, H, do_h, 0, unroll=False)

    return gk


def kernel(input_variable, embedding_weight):
    B, H = input_variable.shape
    V, D = embedding_weight.shape
    idxT = input_variable.astype(jnp.int32).T
    tabP = embedding_weight.reshape(V // 2, 2 * D)
    out = _build(B, H, D)(idxT, tabP)
    return jnp.transpose(out, (2, 0, 1))
